# baseline (speedup 1.0000x reference)
import functools

import jax
import jax.numpy as jnp
from jax import lax
from jax.experimental import pallas as pl
from jax.experimental.pallas import tpu as pltpu
from jax.experimental.pallas import tpu_sc as plsc

_B = 4096
_D = 8192
_NC = 2
_NS = 16
_NW = _NC * _NS
_BW = _B // _NW
_R = 1
_STEPS = _BW // _R
_NBUF = 14
_LOOK = 7

_mesh = plsc.VectorSubcoreMesh(core_axis_name="c", subcore_axis_name="s")


@functools.partial(
    pl.kernel,
    mesh=_mesh,
    out_type=jax.ShapeDtypeStruct((_B, _D), jnp.float32),
    scratch_types=(
        [pltpu.VMEM((_STEPS, _R), jnp.int32)]
        + [pltpu.VMEM((_R, _D), jnp.float32)] * _NBUF
        + [pltpu.SemaphoreType.DMA] * (2 * _NBUF)
    ),
)
def _gather_rows(table_hbm, idx_hbm, out_hbm, idx_v, *bufs_and_sems):
    bufs = bufs_and_sems[:_NBUF]
    gsems = bufs_and_sems[_NBUF:2 * _NBUF]
    ssems = bufs_and_sems[2 * _NBUF:]
    wid = lax.axis_index("s") * _NC + lax.axis_index("c")
    base = wid * _BW
    pltpu.sync_copy(idx_hbm.at[wid], idx_v)

    def gather_start(g, p):
        pltpu.async_copy(table_hbm.at[idx_v.at[g]], bufs[p], gsems[p])

    def gather_wait(p):
        pltpu.make_async_copy(table_hbm.at[idx_v.at[0]], bufs[p],
                              gsems[p]).wait()

    def scatter_start(g, p):
        pltpu.async_copy(bufs[p], out_hbm.at[pl.ds(base + g * _R, _R)],
                         ssems[p])

    def scatter_wait(g, p):
        pltpu.make_async_copy(bufs[p], out_hbm.at[pl.ds(base + g * _R, _R)],
                              ssems[p]).wait()

    for g in range(_LOOK):
        gather_start(g, g % _NBUF)
    for g in range(_STEPS):
        p = g % _NBUF
        gather_wait(p)
        scatter_start(g, p)
        if g + _LOOK < _STEPS:
            d = g + _LOOK - _NBUF
            if d >= 0:
                scatter_wait(d, d % _NBUF)
            gather_start(g + _LOOK, (g + _LOOK) % _NBUF)
    for g in range(_STEPS - _NBUF, _STEPS):
        scatter_wait(g, g % _NBUF)


def kernel(idx, logits):
    idx3 = idx.astype(jnp.int32).reshape(_NW, _STEPS, _R)
    return _gather_rows(logits, idx3)


# speedup vs baseline: 1.0158x; 1.0158x over previous
import functools

import jax
import jax.numpy as jnp
from jax import lax
from jax.experimental import pallas as pl
from jax.experimental.pallas import tpu as pltpu
from jax.experimental.pallas import tpu_sc as plsc

_B = 4096
_D = 8192
_NC = 2
_NS = 16
_NW = _NC * _NS
_BW = _B // _NW
_R = 2
_STEPS = _BW // _R
_NBUF = 7
_LOOK = 6

_mesh = plsc.VectorSubcoreMesh(core_axis_name="c", subcore_axis_name="s")


@functools.partial(
    pl.kernel,
    mesh=_mesh,
    out_type=jax.ShapeDtypeStruct((_B, _D), jnp.float32),
    scratch_types=(
        [pltpu.VMEM((_STEPS, _R), jnp.int32)]
        + [pltpu.VMEM((_R, _D), jnp.float32)] * _NBUF
        + [pltpu.SemaphoreType.DMA] * (2 * _NBUF)
    ),
)
def _gather_rows(table_hbm, idx_hbm, out_hbm, idx_v, *bufs_and_sems):
    bufs = bufs_and_sems[:_NBUF]
    gsems = bufs_and_sems[_NBUF:2 * _NBUF]
    ssems = bufs_and_sems[2 * _NBUF:]
    wid = lax.axis_index("s") * _NC + lax.axis_index("c")
    base = wid * _BW
    pltpu.sync_copy(idx_hbm.at[wid], idx_v)

    def gather_start(g, p):
        pltpu.async_copy(table_hbm.at[idx_v.at[g]], bufs[p], gsems[p])

    def gather_wait(p):
        pltpu.make_async_copy(table_hbm.at[idx_v.at[0]], bufs[p],
                              gsems[p]).wait()

    def scatter_start(g, p):
        pltpu.async_copy(bufs[p], out_hbm.at[pl.ds(base + g * _R, _R)],
                         ssems[p])

    def scatter_wait(g, p):
        pltpu.make_async_copy(bufs[p], out_hbm.at[pl.ds(base + g * _R, _R)],
                              ssems[p]).wait()

    for g in range(_LOOK):
        gather_start(g, g % _NBUF)
    for g in range(_STEPS):
        p = g % _NBUF
        gather_wait(p)
        scatter_start(g, p)
        if g + _LOOK < _STEPS:
            d = g + _LOOK - _NBUF
            if d >= 0:
                scatter_wait(d, d % _NBUF)
            gather_start(g + _LOOK, (g + _LOOK) % _NBUF)
    for g in range(_STEPS - _NBUF, _STEPS):
        scatter_wait(g, g % _NBUF)


def kernel(idx, logits):
    idx3 = idx.astype(jnp.int32).reshape(_NW, _STEPS, _R)
    return _gather_rows(logits, idx3)
